# baseline (speedup 1.0000x reference)
import functools

import jax
import jax.numpy as jnp
from jax import lax
from jax.experimental import pallas as pl
from jax.experimental.pallas import tpu as pltpu
from jax.experimental.pallas import tpu_sc as plsc

B = 16384
FW = 20
ED = 32
VOCAB = 1000000
NC = 2
NS = 16
NW = NC * NS
RPW = B // NW
NG = RPW * FW // 128
PK = 8
NP = VOCAB // PK
BN8 = 2048
GRID = (NP + BN8 - 1) // BN8
SLEN = GRID * BN8 * PK


def _mm_body(w2_ref, t_ref, *out_refs):
    sa = []
    for a in range(PK):
        ta = t_ref[:, pl.ds(a * ED, ED)]
        sa.append(lax.dot_general(
            w2_ref[...], ta,
            dimension_numbers=(((1,), (1,)), ((), ())),
            preferred_element_type=jnp.float32))
    for f in range(FW):
        out_refs[f][...] = jnp.concatenate([sa[a][f] for a in range(PK)])


@jax.jit
def _scores(t8, w2):
    return pl.pallas_call(
        _mm_body,
        grid=(GRID,),
        in_specs=[
            pl.BlockSpec((FW, ED), lambda i: (0, 0)),
            pl.BlockSpec((BN8, PK * ED), lambda i: (i, 0)),
        ],
        out_specs=[pl.BlockSpec((BN8 * PK,), lambda i: (i,))
                   for _ in range(FW)],
        out_shape=[jax.ShapeDtypeStruct((SLEN,), jnp.float32)
                   for _ in range(FW)],
    )(w2, t8)


def _sc_body(*refs):
    s_hbm = refs[:FW]
    (xidx_hbm, bias_hbm, out_hbm,
     idx_v, vals_v, bias_v, out_v, sem) = refs[FW:]
    wid = lax.axis_index("s") * NC + lax.axis_index("c")

    pltpu.sync_copy(bias_hbm, bias_v)
    bv = bias_v[pl.ds(0, 16)]

    pltpu.sync_copy(xidx_hbm.at[wid], idx_v)
    cps = [
        pltpu.async_copy(s_hbm[j // 4].at[idx_v.at[j]], vals_v.at[j], sem)
        for j in range(NG)
    ]
    for cp in cps:
        cp.wait()

    for ob in range(RPW // 16):
        acc = vals_v[ob // 8, pl.ds((ob % 8) * 16, 16)]
        for f in range(1, FW):
            acc = acc + vals_v[f * 4 + ob // 8, pl.ds((ob % 8) * 16, 16)]
        out_v[pl.ds(ob * 16, 16)] = jnp.exp(acc + bv)

    pltpu.sync_copy(out_v, out_hbm.at[pl.ds(wid * RPW, RPW)])


@jax.jit
def _run(s_list, xi, bias):
    mesh = plsc.VectorSubcoreMesh(core_axis_name="c", subcore_axis_name="s")
    return pl.kernel(
        _sc_body,
        mesh=mesh,
        out_type=jax.ShapeDtypeStruct((B,), jnp.float32),
        scratch_types=[
            pltpu.VMEM((NG, 128), jnp.int32),
            pltpu.VMEM((NG, 128), jnp.float32),
            pltpu.VMEM((128,), jnp.float32),
            pltpu.VMEM((RPW,), jnp.float32),
            pltpu.SemaphoreType.DMA,
        ],
    )(*s_list, xi, bias)


def kernel(x, table, W, b):
    w2 = W.astype(jnp.float32).reshape(FW, ED)
    t8 = table.reshape(NP, PK * ED)
    s_list = _scores(t8, w2)
    xo = x.astype(jnp.int32).reshape(NW, RPW, FW).transpose(0, 2, 1)
    q = xo // PK
    xo = (q // BN8) * (BN8 * PK) + (xo % PK) * BN8 + q % BN8
    xi = xo.reshape(NW, NG, 128)
    bias = jnp.broadcast_to(b.astype(jnp.float32), (128,))
    return _run(s_list, xi, bias)


# speedup vs baseline: 1.2816x; 1.2816x over previous
import functools

import jax
import jax.numpy as jnp
from jax import lax
from jax.experimental import pallas as pl
from jax.experimental.pallas import tpu as pltpu
from jax.experimental.pallas import tpu_sc as plsc

B = 16384
FW = 20
ED = 32
VOCAB = 1000000
NC = 2
NS = 16
NW = NC * NS
RPW = B // NW
NG = RPW * FW // 128
BN = 32768


def _mm_body(w2_ref, t_ref, *out_refs):
    mm = lax.dot_general(
        w2_ref[...], t_ref[...],
        dimension_numbers=(((1,), (1,)), ((), ())),
        preferred_element_type=jnp.float32)
    for f in range(FW):
        out_refs[f][...] = mm[f]


@jax.jit
def _scores(table, w2):
    return pl.pallas_call(
        _mm_body,
        grid=((VOCAB + BN - 1) // BN,),
        in_specs=[
            pl.BlockSpec((FW, ED), lambda i: (0, 0)),
            pl.BlockSpec((BN, ED), lambda i: (i, 0)),
        ],
        out_specs=[pl.BlockSpec((BN,), lambda i: (i,))
                   for _ in range(FW)],
        out_shape=[jax.ShapeDtypeStruct((VOCAB,), jnp.float32)
                   for _ in range(FW)],
    )(w2, table)


def _sc_body(*refs):
    s_hbm = refs[:FW]
    (xidx_hbm, bias_hbm, out_hbm,
     idx_v, vals_v, bias_v, out_v, sem) = refs[FW:]
    wid = lax.axis_index("s") * NC + lax.axis_index("c")

    pltpu.sync_copy(bias_hbm, bias_v)
    bv = bias_v[pl.ds(0, 16)]

    pltpu.sync_copy(xidx_hbm.at[wid], idx_v)
    cps = [
        pltpu.async_copy(s_hbm[j // 4].at[idx_v.at[j]], vals_v.at[j], sem)
        for j in range(NG)
    ]
    for cp in cps:
        cp.wait()

    for ob in range(RPW // 16):
        acc = vals_v[ob // 8, pl.ds((ob % 8) * 16, 16)]
        for f in range(1, FW):
            acc = acc + vals_v[f * 4 + ob // 8, pl.ds((ob % 8) * 16, 16)]
        out_v[pl.ds(ob * 16, 16)] = jnp.exp(acc + bv)

    pltpu.sync_copy(out_v, out_hbm.at[pl.ds(wid * RPW, RPW)])


@jax.jit
def _run(s_list, xi, bias):
    mesh = plsc.VectorSubcoreMesh(core_axis_name="c", subcore_axis_name="s")
    return pl.kernel(
        _sc_body,
        mesh=mesh,
        out_type=jax.ShapeDtypeStruct((B,), jnp.float32),
        scratch_types=[
            pltpu.VMEM((NG, 128), jnp.int32),
            pltpu.VMEM((NG, 128), jnp.float32),
            pltpu.VMEM((128,), jnp.float32),
            pltpu.VMEM((RPW,), jnp.float32),
            pltpu.SemaphoreType.DMA,
        ],
    )(*s_list, xi, bias)


def kernel(x, table, W, b):
    w2 = W.astype(jnp.float32).reshape(FW, ED)
    s_list = _scores(table, w2)
    xo = x.astype(jnp.int32).reshape(NW, RPW, FW).transpose(0, 2, 1)
    xi = xo.reshape(NW, NG, 128)
    bias = jnp.broadcast_to(b.astype(jnp.float32), (128,))
    return _run(s_list, xi, bias)
